# baseline (speedup 1.0000x reference)
import jax
import jax.numpy as jnp
from jax.experimental import pallas as pl
from jax.experimental.pallas import tpu as pltpu

B = 64
L = 2048
D = 128
_WORDS = L * D


def _body(offs_smem, pe_hbm, out_hbm, sem):
    def issue(b, _):
        off = offs_smem[b]
        pltpu.make_async_copy(
            pe_hbm.at[pl.ds(off * D, _WORDS)],
            out_hbm.at[pl.ds(b * _WORDS, _WORDS)],
            sem,
        ).start()
        return 0

    jax.lax.fori_loop(0, B, issue, 0)

    def drain(b, _):
        pltpu.make_async_copy(
            pe_hbm.at[pl.ds(0, _WORDS)],
            out_hbm.at[pl.ds(b * _WORDS, _WORDS)],
            sem,
        ).wait()
        return 0

    jax.lax.fori_loop(0, B, drain, 0)


def kernel(x, pe):
    offsets = x[:, 0, 0].astype(jnp.int32)
    grid_spec = pltpu.PrefetchScalarGridSpec(
        num_scalar_prefetch=1,
        grid=(1,),
        in_specs=[pl.BlockSpec(memory_space=pltpu.MemorySpace.HBM)],
        out_specs=pl.BlockSpec(memory_space=pltpu.MemorySpace.HBM),
        scratch_shapes=[pltpu.SemaphoreType.DMA],
    )
    flat = pl.pallas_call(
        _body,
        grid_spec=grid_spec,
        out_shape=jax.ShapeDtypeStruct((B * _WORDS,), jnp.float32),
    )(offsets, pe.reshape(-1))
    return flat.reshape(B, L, D)


# speedup vs baseline: 67.0951x; 67.0951x over previous
import jax
import jax.numpy as jnp
from jax.experimental import pallas as pl
from jax.experimental.pallas import tpu as pltpu

B = 64
L = 2048
D = 128
_WORDS = L * D
_PE_WORDS = 30720 * 128


def _body(offs_smem, pe_hbm, out_hbm, pe_vmem, sem_in, sem):
    pltpu.make_async_copy(pe_hbm, pe_vmem, sem_in).start()
    pltpu.make_async_copy(pe_hbm, pe_vmem, sem_in).wait()

    def issue(b, _):
        off = offs_smem[b]
        pltpu.make_async_copy(
            pe_vmem.at[pl.ds(off * D, _WORDS)],
            out_hbm.at[pl.ds(b * _WORDS, _WORDS)],
            sem,
        ).start()
        return 0

    jax.lax.fori_loop(0, B, issue, 0)

    def drain(b, _):
        pltpu.make_async_copy(
            pe_vmem.at[pl.ds(0, _WORDS)],
            out_hbm.at[pl.ds(b * _WORDS, _WORDS)],
            sem,
        ).wait()
        return 0

    jax.lax.fori_loop(0, B, drain, 0)


def kernel(x, pe):
    offsets = x[:, 0, 0].astype(jnp.int32)
    grid_spec = pltpu.PrefetchScalarGridSpec(
        num_scalar_prefetch=1,
        grid=(1,),
        in_specs=[pl.BlockSpec(memory_space=pltpu.MemorySpace.HBM)],
        out_specs=pl.BlockSpec(memory_space=pltpu.MemorySpace.HBM),
        scratch_shapes=[
            pltpu.VMEM((_PE_WORDS,), jnp.float32),
            pltpu.SemaphoreType.DMA,
            pltpu.SemaphoreType.DMA,
        ],
    )
    flat = pl.pallas_call(
        _body,
        grid_spec=grid_spec,
        out_shape=jax.ShapeDtypeStruct((B * _WORDS,), jnp.float32),
    )(offsets, pe.reshape(-1))
    return flat.reshape(B, L, D)
